# Initial kernel scaffold; baseline (speedup 1.0000x reference)
import functools

import jax
import jax.numpy as jnp
from jax import lax
from jax.experimental import pallas as pl
from jax.experimental.pallas import tpu as pltpu
from jax.experimental.pallas import tpu_sc as plsc

VOCAB_ = 1000000
DIM_ = 32
WPR = DIM_ // 4
NC = 2
NS = 16
NW = NC * NS
C = 128
LANES = 16


def _body(n_chunks, w_hbm, x_hbm, scale_hbm, out_hbm, idx_v, rows_v, out_v,
          scale_v, sem):
    wid = lax.axis_index("s") * NC + lax.axis_index("c")
    pltpu.sync_copy(scale_hbm, scale_v)
    s = scale_v[...]

    iota = lax.iota(jnp.int32, LANES)
    hi3 = lax.shift_right_logical(iota, 3)
    lo3 = lax.bitwise_and(iota, 7)
    i4 = iota * 4

    pltpu.sync_copy(x_hbm.at[pl.ds(wid * n_chunks, n_chunks)], idx_v)

    def chunk(i, carry):
        pltpu.async_copy(w_hbm.at[idx_v.at[i]], rows_v, sem).wait()
        for j in range(C * WPR // LANES):
            r_idx = hi3 + (j * 2)
            w = plsc.load_gather(rows_v, [r_idx, lo3])
            base = j * (LANES * 4)
            for k in range(4):
                if k < 3:
                    b = lax.shift_right_arithmetic(
                        lax.shift_left(w, 24 - 8 * k), 24)
                else:
                    b = lax.shift_right_arithmetic(w, 24)
                f = b.astype(jnp.float32) * s
                plsc.store_scatter(out_v, [i4 + (base + k)], f)
        pltpu.sync_copy(
            out_v, out_hbm.at[pl.ds((wid * n_chunks + i) * (C * DIM_),
                                    C * DIM_)])
        return carry

    lax.fori_loop(0, n_chunks, chunk, 0)


def kernel(x, weight, scale):
    B, L = x.shape
    vocab, dim = weight.shape
    n = B * L
    assert dim == DIM_ and n % (NW * C) == 0
    n_chunks = n // (NW * C)

    w32 = lax.bitcast_convert_type(weight.reshape(vocab, WPR, 4), jnp.int32)
    x2 = x.reshape(n // C, C)
    scale_vec = jnp.broadcast_to(scale.astype(jnp.float32), (LANES,))

    mesh = plsc.VectorSubcoreMesh(core_axis_name="c", subcore_axis_name="s",
                                  num_cores=NC, num_subcores=NS)
    kern = pl.kernel(
        functools.partial(_body, n_chunks),
        out_type=jax.ShapeDtypeStruct((n * dim,), jnp.float32),
        mesh=mesh,
        scratch_types=[
            pltpu.VMEM((n_chunks, C), jnp.int32),
            pltpu.VMEM((C, WPR), jnp.int32),
            pltpu.VMEM((C * DIM_,), jnp.float32),
            pltpu.VMEM((LANES,), jnp.float32),
            pltpu.SemaphoreType.DMA,
        ],
    )
    out = kern(w32, x2, scale_vec)
    return out.reshape(B, L, dim)


# speedup vs baseline: 4.4902x; 4.4902x over previous
import functools

import jax
import jax.numpy as jnp
from jax import lax
from jax.experimental import pallas as pl
from jax.experimental.pallas import tpu as pltpu
from jax.experimental.pallas import tpu_sc as plsc

VOCAB_ = 1000000
DIM_ = 32
WPR = DIM_ // 4
NC = 2
NS = 16
NW = NC * NS
C = 128
LANES = 16


def _body(n_chunks, w_hbm, x_hbm, scale_hbm, out_hbm, idx_v, rows_v, out_v,
          scale_v, sem):
    wid = lax.axis_index("s") * NC + lax.axis_index("c")
    pltpu.sync_copy(scale_hbm, scale_v)
    s = scale_v[...]

    iota = lax.iota(jnp.int32, LANES)
    i4 = iota * 4
    hi3 = lax.shift_right_logical(iota, 3)
    lo3 = lax.bitwise_and(iota, 7)

    pltpu.sync_copy(x_hbm.at[pl.ds(wid * n_chunks, n_chunks)], idx_v)

    def chunk(i, carry):
        pltpu.async_copy(w_hbm.at[idx_v.at[i]], rows_v, sem).wait()
        for j in range(C * WPR // LANES):
            w = plsc.load_gather(rows_v, [hi3 + (j * 2), lo3])
            base = j * (LANES * 4)
            for k in range(4):
                if k < 3:
                    b = lax.shift_right_arithmetic(
                        lax.shift_left(w, 24 - 8 * k), 24)
                else:
                    b = lax.shift_right_arithmetic(w, 24)
                f = b.astype(jnp.float32) * s
                plsc.store_scatter(out_v, [i4 + (base + k)], f)
        pltpu.sync_copy(
            out_v, out_hbm.at[pl.ds((wid * n_chunks + i) * (C * DIM_),
                                    C * DIM_)])
        return carry

    lax.fori_loop(0, n_chunks, chunk, 0)


def kernel(x, weight, scale):
    B, L = x.shape
    vocab, dim = weight.shape
    n = B * L
    assert dim == DIM_ and n % (NW * C) == 0
    n_chunks = n // (NW * C)

    w32 = lax.bitcast_convert_type(weight.reshape(vocab, WPR, 4), jnp.int32)
    x2 = x.reshape(n // C, C)
    scale_vec = jnp.broadcast_to(scale.astype(jnp.float32), (LANES,))

    mesh = plsc.VectorSubcoreMesh(core_axis_name="c", subcore_axis_name="s",
                                  num_cores=NC, num_subcores=NS)
    kern = pl.kernel(
        functools.partial(_body, n_chunks),
        out_type=jax.ShapeDtypeStruct((n * dim,), jnp.float32),
        mesh=mesh,
        scratch_types=[
            pltpu.VMEM((n_chunks, C), jnp.int32),
            pltpu.VMEM((C, WPR), jnp.int32),
            pltpu.VMEM((C * DIM_,), jnp.float32),
            pltpu.VMEM((LANES,), jnp.float32),
            pltpu.SemaphoreType.DMA,
        ],
        compiler_params=pltpu.CompilerParams(needs_layout_passes=False,
                                             use_tc_tiling_on_sc=False),
    )
    out = kern(w32, x2, scale_vec)
    return out.reshape(B, L, dim)


# speedup vs baseline: 6.9605x; 1.5502x over previous
import functools

import jax
import jax.numpy as jnp
from jax import lax
from jax.experimental import pallas as pl
from jax.experimental.pallas import tpu as pltpu
from jax.experimental.pallas import tpu_sc as plsc

VOCAB_ = 1000000
DIM_ = 32
WPR = DIM_ // 4
NC = 2
NS = 16
NW = NC * NS
C = 128
LANES = 16


def _body(n_chunks, w_hbm, x_hbm, scale_hbm, out_hbm, idx_v, rows_v, out_v,
          scale_v, sem):
    wid = lax.axis_index("s") * NC + lax.axis_index("c")
    pltpu.sync_copy(scale_hbm, scale_v)
    s = scale_v[...]

    iota = lax.iota(jnp.int32, LANES)
    i4 = iota * 4
    rows_w = rows_v.bitcast(jnp.int32)

    pltpu.sync_copy(x_hbm.at[pl.ds(wid * n_chunks, n_chunks)], idx_v)

    def chunk(i, carry):
        pltpu.async_copy(w_hbm.at[idx_v.at[i]], rows_v, sem).wait()
        for j in range(C * WPR // LANES):
            w = rows_w[j >> 1, pl.ds((j & 1) * LANES, LANES)]
            base = j * (LANES * 4)
            for k in range(4):
                if k < 3:
                    b = lax.shift_right_arithmetic(
                        lax.shift_left(w, 24 - 8 * k), 24)
                else:
                    b = lax.shift_right_arithmetic(w, 24)
                f = b.astype(jnp.float32) * s
                plsc.store_scatter(out_v, [i4 + (base + k)], f)
        pltpu.sync_copy(
            out_v, out_hbm.at[pl.ds((wid * n_chunks + i) * (C * DIM_),
                                    C * DIM_)])
        return carry

    lax.fori_loop(0, n_chunks, chunk, 0)


def kernel(x, weight, scale):
    B, L = x.shape
    vocab, dim = weight.shape
    n = B * L
    assert dim == DIM_ and n % (NW * C) == 0
    n_chunks = n // (NW * C)

    x2 = x.reshape(n // C, C)
    scale_vec = jnp.broadcast_to(scale.astype(jnp.float32), (LANES,))

    mesh = plsc.VectorSubcoreMesh(core_axis_name="c", subcore_axis_name="s",
                                  num_cores=NC, num_subcores=NS)
    kern = pl.kernel(
        functools.partial(_body, n_chunks),
        out_type=jax.ShapeDtypeStruct((n * dim,), jnp.float32),
        mesh=mesh,
        scratch_types=[
            pltpu.VMEM((n_chunks, C), jnp.int32),
            pltpu.VMEM((C, DIM_), jnp.int8),
            pltpu.VMEM((C * DIM_,), jnp.float32),
            pltpu.VMEM((LANES,), jnp.float32),
            pltpu.SemaphoreType.DMA,
        ],
        compiler_params=pltpu.CompilerParams(needs_layout_passes=False,
                                             use_tc_tiling_on_sc=False),
    )
    out = kern(weight, x2, scale_vec)
    return out.reshape(B, L, dim)


# speedup vs baseline: 7.0748x; 1.0164x over previous
import functools

import jax
import jax.numpy as jnp
from jax import lax
from jax.experimental import pallas as pl
from jax.experimental.pallas import tpu as pltpu
from jax.experimental.pallas import tpu_sc as plsc

DIM_ = 32
NC = 2
NS = 16
NW = NC * NS
C = 128
G = 4


def _gather_body(n_steps, w_hbm, x_hbm, out_hbm, idx_v, raw_v, sems):
    wid = lax.axis_index("s") * NC + lax.axis_index("c")
    pltpu.sync_copy(x_hbm.at[pl.ds(wid * n_steps * G, n_steps * G)], idx_v)

    def step(i, carry):
        copies = []
        for g in range(G):
            copies.append(pltpu.make_async_copy(
                w_hbm.at[idx_v.at[i * G + g]],
                raw_v.at[pl.ds(g * C, C)], sems.at[g]))
        for c in copies:
            c.start()
        for c in copies:
            c.wait()
        pltpu.sync_copy(
            raw_v, out_hbm.at[pl.ds((wid * n_steps + i) * (G * C), G * C)])
        return carry

    lax.fori_loop(0, n_steps, step, 0)


def _dequant_body(x_ref, s_ref, o_ref):
    o_ref[...] = x_ref[...].astype(jnp.float32) * s_ref[0, 0]


def kernel(x, weight, scale):
    B, L = x.shape
    vocab, dim = weight.shape
    n = B * L
    assert dim == DIM_ and n % (NW * C * G) == 0
    n_steps = n // (NW * C * G)

    x2 = x.reshape(n // C, C)

    mesh = plsc.VectorSubcoreMesh(core_axis_name="c", subcore_axis_name="s",
                                  num_cores=NC, num_subcores=NS)
    gather = pl.kernel(
        functools.partial(_gather_body, n_steps),
        out_type=jax.ShapeDtypeStruct((n, dim), jnp.int8),
        mesh=mesh,
        scratch_types=[
            pltpu.VMEM((n_steps * G, C), jnp.int32),
            pltpu.VMEM((G * C, dim), jnp.int8),
            pltpu.SemaphoreType.DMA((G,)),
        ],
        compiler_params=pltpu.CompilerParams(needs_layout_passes=False,
                                             use_tc_tiling_on_sc=False),
    )
    g8 = gather(weight, x2)

    rows_blk = 128
    g2 = g8.reshape(n * dim // 4096, 4096)
    dequant = pl.pallas_call(
        _dequant_body,
        grid=(g2.shape[0] // rows_blk,),
        in_specs=[
            pl.BlockSpec((rows_blk, 4096), lambda i: (i, 0)),
            pl.BlockSpec(memory_space=pltpu.SMEM),
        ],
        out_specs=pl.BlockSpec((rows_blk, 4096), lambda i: (i, 0)),
        out_shape=jax.ShapeDtypeStruct(g2.shape, jnp.float32),
    )
    out = dequant(g2, scale.astype(jnp.float32).reshape(1, 1))
    return out.reshape(B, L, dim)


# speedup vs baseline: 7.1871x; 1.0159x over previous
import functools

import jax
import jax.numpy as jnp
from jax import lax
from jax.experimental import pallas as pl
from jax.experimental.pallas import tpu as pltpu
from jax.experimental.pallas import tpu_sc as plsc

DIM_ = 32
NC = 2
NS = 16
NW = NC * NS
C = 128
G = 4


def _gather_body(n_steps, w_hbm, x_hbm, out_hbm, idx_v, raw_v, sems):
    wid = lax.axis_index("s") * NC + lax.axis_index("c")
    pltpu.sync_copy(x_hbm.at[pl.ds(wid * n_steps * G, n_steps * G)], idx_v)

    def step(i, carry):
        copies = []
        for g in range(G):
            copies.append(pltpu.make_async_copy(
                w_hbm.at[idx_v.at[i * G + g]],
                raw_v.at[pl.ds(g * C, C)], sems.at[g]))
        for c in copies:
            c.start()
        for c in copies:
            c.wait()
        pltpu.sync_copy(
            raw_v, out_hbm.at[pl.ds((wid * n_steps + i) * (G * C), G * C)])
        return carry

    lax.fori_loop(0, n_steps, step, 0)


def _dequant_body(L, x_ref, s_ref, o_ref):
    v = x_ref[...].astype(jnp.float32) * s_ref[0, 0]
    b_blk = x_ref.shape[0]
    o_ref[...] = v.reshape(b_blk, L, DIM_).transpose(1, 2, 0)


def kernel(x, weight, scale):
    B, L = x.shape
    vocab, dim = weight.shape
    n = B * L
    assert dim == DIM_ and n % (NW * C * G) == 0
    n_steps = n // (NW * C * G)

    x2 = x.reshape(n // C, C)

    mesh = plsc.VectorSubcoreMesh(core_axis_name="c", subcore_axis_name="s",
                                  num_cores=NC, num_subcores=NS)
    gather = pl.kernel(
        functools.partial(_gather_body, n_steps),
        out_type=jax.ShapeDtypeStruct((n, dim), jnp.int8),
        mesh=mesh,
        scratch_types=[
            pltpu.VMEM((n_steps * G, C), jnp.int32),
            pltpu.VMEM((G * C, dim), jnp.int8),
            pltpu.SemaphoreType.DMA((G,)),
        ],
        compiler_params=pltpu.CompilerParams(needs_layout_passes=False,
                                             use_tc_tiling_on_sc=False),
    )
    g8 = gather(weight, x2)

    b_blk = 1024
    g3 = g8.reshape(B, L * dim)
    dequant = pl.pallas_call(
        functools.partial(_dequant_body, L),
        grid=(B // b_blk,),
        in_specs=[
            pl.BlockSpec((b_blk, L * dim), lambda i: (i, 0)),
            pl.BlockSpec(memory_space=pltpu.SMEM),
        ],
        out_specs=pl.BlockSpec((L, dim, b_blk), lambda i: (0, 0, i)),
        out_shape=jax.ShapeDtypeStruct((L, dim, B), jnp.float32),
    )
    out = dequant(g3, scale.astype(jnp.float32).reshape(1, 1))
    return out.transpose(2, 0, 1)
